# Initial kernel scaffold; baseline (speedup 1.0000x reference)
import jax
import jax.numpy as jnp
from jax.experimental import pallas as pl


def kernel(x, res_w_0, res_b_0, res_g_0, res_be_0, res_w_1, res_b_1, res_g_1, res_be_1, down_w, down_b, down_g, down_be):
    raise NotImplementedError("write your pallas kernel here")


# speedup vs baseline: 22.3820x; 22.3820x over previous
import jax
import jax.numpy as jnp
from jax.experimental import pallas as pl
from jax.experimental.pallas import tpu as pltpu

LEAKY_SLOPE = 0.01
BN_EPS = 1e-5
IPB = 2


def _prep_w(w_oihw):
    o, i, kh, kw = w_oihw.shape
    w = jnp.transpose(w_oihw, (2, 3, 1, 0)).reshape(kh, kw * i, o)
    return w.astype(jnp.bfloat16)


def _lrelu(z):
    return jnp.where(z >= 0, z, LEAKY_SLOPE * z)


def _conv3x3_s1(a, w_ref):
    b, H, W, C = a.shape
    ab = a.astype(jnp.bfloat16)
    ap = jnp.pad(ab, ((0, 0), (1, 1), (1, 1), (0, 0)))
    xc = jnp.concatenate(
        [ap[:, :, 0:W, :], ap[:, :, 1:W + 1, :], ap[:, :, 2:W + 2, :]],
        axis=3)
    m = b * H * W
    acc = jnp.dot(xc[:, 0:H].reshape(m, 3 * C), w_ref[0],
                  preferred_element_type=jnp.float32)
    acc = acc + jnp.dot(xc[:, 1:H + 1].reshape(m, 3 * C), w_ref[1],
                        preferred_element_type=jnp.float32)
    acc = acc + jnp.dot(xc[:, 2:H + 2].reshape(m, 3 * C), w_ref[2],
                        preferred_element_type=jnp.float32)
    return acc


def _emit(y, y_ref, s_ref, q_ref):
    y_ref[...] = y.reshape(y_ref.shape)
    yr = y.reshape(-1, 8, y.shape[-1])
    s_ref[0] = jnp.sum(yr, axis=0)
    q_ref[0] = jnp.sum(yr * yr, axis=0)


def _k_first(x_ref, w_ref, y_ref, s_ref, q_ref):
    _emit(_conv3x3_s1(x_ref[...], w_ref), y_ref, s_ref, q_ref)


def _k_mid(yp_ref, w_ref, ss_ref, y_ref, s_ref, q_ref):
    a = _lrelu(yp_ref[...] * ss_ref[0] + ss_ref[1])
    _emit(_conv3x3_s1(a, w_ref), y_ref, s_ref, q_ref)


def _k_mid_res(yp_ref, r_ref, w_ref, ss_ref, y_ref, s_ref, q_ref):
    a = r_ref[...] + _lrelu(yp_ref[...] * ss_ref[0] + ss_ref[1])
    _emit(_conv3x3_s1(a, w_ref), y_ref, s_ref, q_ref)


def _k_down(y4_ref, y2_ref, x0_ref, w_ref, ss2_ref, ss4_ref,
            y_ref, s_ref, q_ref):
    a = (x0_ref[...]
         + _lrelu(y2_ref[...] * ss2_ref[0] + ss2_ref[1])
         + _lrelu(y4_ref[...] * ss4_ref[0] + ss4_ref[1]))
    b, H, W, C = a.shape
    Ho, Wo = H // 2, W // 2
    ab = a.astype(jnp.bfloat16)
    ap = jnp.pad(ab, ((0, 0), (1, 1), (1, 1), (0, 0)))
    pw = ap[:, :, 0:W, :].reshape(b, H + 2, Wo, 2, C)
    pw2 = ap[:, :, 2:W + 2, :].reshape(b, H + 2, Wo, 2, C)
    xc = jnp.concatenate(
        [pw[:, :, :, 0, :],
         pw[:, :, :, 1, :],
         pw2[:, :, :, 0, :]],
        axis=3)
    m = b * Ho * Wo
    acc = None
    for dy in range(3):
        lhs = xc[:, dy:dy + H, :, :].reshape(b, Ho, 2, Wo, 3 * C)[:, :, 0]
        d = jnp.dot(lhs.reshape(m, 3 * C), w_ref[dy],
                    preferred_element_type=jnp.float32)
        acc = d if acc is None else acc + d
    _emit(acc, y_ref, s_ref, q_ref)


def _k_bn(y_ref, ss_ref, o_ref):
    o_ref[...] = _lrelu(y_ref[...] * ss_ref[0] + ss_ref[1])


def _coeffs(s, q, g, be, count):
    c = s.shape[-1]
    mean = jnp.sum(s.reshape(-1, c), axis=0) / count
    var = jnp.sum(q.reshape(-1, c), axis=0) / count - mean * mean
    var = jnp.maximum(var, 0.0)
    scale = g * jax.lax.rsqrt(var + BN_EPS)
    return jnp.stack([scale, be - mean * scale])


def _img_spec(h, w, c):
    return pl.BlockSpec((IPB, h, w, c), lambda i: (i, 0, 0, 0))


def _full3_spec(shape):
    return pl.BlockSpec(shape, lambda i: (0, 0, 0))


def _ss_spec(c):
    return pl.BlockSpec((2, c), lambda i: (0, 0))


_CP = pltpu.CompilerParams(dimension_semantics=("parallel",),
                           vmem_limit_bytes=100 * 1024 * 1024)


def _conv_call(body, ins, in_specs, n, ho, wo, cout):
    g = n // IPB
    out_shape = (jax.ShapeDtypeStruct((n, ho, wo, cout), jnp.float32),
                 jax.ShapeDtypeStruct((g, 8, cout), jnp.float32),
                 jax.ShapeDtypeStruct((g, 8, cout), jnp.float32))
    out_specs = (_img_spec(ho, wo, cout),
                 pl.BlockSpec((1, 8, cout), lambda i: (i, 0, 0)),
                 pl.BlockSpec((1, 8, cout), lambda i: (i, 0, 0)))
    return pl.pallas_call(
        body, out_shape=out_shape, grid=(g,), in_specs=in_specs,
        out_specs=out_specs, compiler_params=_CP)(*ins)


def kernel(x, res_w_0, res_b_0, res_g_0, res_be_0,
           res_w_1, res_b_1, res_g_1, res_be_1,
           down_w, down_b, down_g, down_be):
    n, c, h, w = x.shape
    cd = down_w.shape[0]
    ho, wo = h // 2, w // 2
    m1 = n * h * w
    m5 = n * ho * wo

    x0 = jnp.transpose(x, (0, 2, 3, 1))
    w0 = _prep_w(res_w_0)
    w1 = _prep_w(res_w_1)
    wd = _prep_w(down_w)

    img = _img_spec(h, w, c)
    wsp = _full3_spec((3, 3 * c, c))
    ssp = _ss_spec(c)

    y1, s1, q1 = _conv_call(_k_first, (x0, w0), [img, wsp], n, h, w, c)
    ss1 = _coeffs(s1, q1, res_g_0, res_be_0, m1)
    y2, s2, q2 = _conv_call(_k_mid, (y1, w0, ss1), [img, wsp, ssp],
                            n, h, w, c)
    ss2 = _coeffs(s2, q2, res_g_0, res_be_0, m1)
    y3, s3, q3 = _conv_call(_k_mid_res, (y2, x0, w1, ss2),
                            [img, img, wsp, ssp], n, h, w, c)
    ss3 = _coeffs(s3, q3, res_g_1, res_be_1, m1)
    y4, s4, q4 = _conv_call(_k_mid, (y3, w1, ss3), [img, wsp, ssp],
                            n, h, w, c)
    ss4 = _coeffs(s4, q4, res_g_1, res_be_1, m1)
    y5, s5, q5 = _conv_call(_k_down, (y4, y2, x0, wd, ss2, ss4),
                            [img, img, img, _full3_spec((3, 3 * c, cd)),
                             ssp, ssp], n, ho, wo, cd)
    ss5 = _coeffs(s5, q5, down_g, down_be, m5)

    out = pl.pallas_call(
        _k_bn,
        out_shape=jax.ShapeDtypeStruct((n, ho, wo, cd), jnp.float32),
        grid=(n // IPB,),
        in_specs=[_img_spec(ho, wo, cd), _ss_spec(cd)],
        out_specs=_img_spec(ho, wo, cd),
        compiler_params=_CP)(y5, ss5)

    return jnp.transpose(out, (0, 3, 1, 2))
